# baseline (speedup 1.0000x reference)
import functools
import jax
import jax.numpy as jnp
from jax import lax
from jax.experimental import pallas as pl
from jax.experimental.pallas import tpu as pltpu

S = 1024
C = 1024
N_PAD = S * C
K = 1024


def _cswap(v, idx, j, stage_k, col_lt):
    g = v.shape[0] // (2 * j)
    c = v.shape[1]
    one = jnp.int32(1)
    if j == 1:
        v3 = v.reshape(g, 2, c)
        i3 = idx.reshape(g, 2, c)
        a_v, b_v = v3[:, 0], v3[:, 1]
        a_i, b_i = i3[:, 0], i3[:, 1]
        row_blk = lax.broadcasted_iota(jnp.int32, (g, 1), 0) * (2 * j)
        col = lax.broadcasted_iota(jnp.int32, (1, c), 1)
    else:
        v4 = v.reshape(g, 2, j, c)
        i4 = idx.reshape(g, 2, j, c)
        a_v, b_v = v4[:, 0], v4[:, 1]
        a_i, b_i = i4[:, 0], i4[:, 1]
        row_blk = lax.broadcasted_iota(jnp.int32, (g, 1, 1), 0) * (2 * j)
        col = lax.broadcasted_iota(jnp.int32, (1, 1) + (c,), len((1, 1)))
    if stage_k:
        stage_desc = jnp.where((row_blk & stage_k) == 0, one, 0)
    else:
        stage_desc = jnp.full_like(row_blk, 1)
    col_desc = jnp.where(col < col_lt, one, 0)
    dir_desc = stage_desc * col_desc + (1 - stage_desc) * (1 - col_desc)
    aw = (a_v > b_v) | ((a_v == b_v) & (a_i < b_i))
    aw_i = jnp.where(aw, one, 0)
    keep = aw_i == dir_desc
    hi_v = jnp.where(keep, a_v, b_v)
    lo_v = jnp.where(keep, b_v, a_v)
    hi_i = jnp.where(keep, a_i, b_i)
    lo_i = jnp.where(keep, b_i, a_i)
    v = jnp.stack([hi_v, lo_v], axis=1).reshape(2 * g * j, c)
    idx = jnp.stack([hi_i, lo_i], axis=1).reshape(2 * g * j, c)
    return v, idx


CB = 128
NSTEPS = C // CB


def _colsort_kernel(imp_ref, vals_ref, idx_ref):
    s = pl.program_id(0)
    v = imp_ref[...]
    idx = (lax.broadcasted_iota(jnp.int32, (S, CB), 0) * C
           + lax.broadcasted_iota(jnp.int32, (S, CB), 1) + s * CB)

    col_lt = jnp.where(s < NSTEPS // 2, CB, 0)
    k = 2
    while k <= S:
        j = k // 2
        while j >= 1:
            v, idx = _cswap(v, idx, j, k, col_lt)
            j //= 2
        k *= 2
    vals_ref[...] = v
    idx_ref[...] = idx


def _fold_kernel(thr_ref, cvals_ref, cidx_ref, ovals_ref, oidx_ref, omask_ref):
    v = cvals_ref[...]
    idx = cidx_ref[...]

    w = C
    while w > 1:
        half = w // 2
        a_v, b_v = v[:, :half], v[:, half:w]
        a_i, b_i = idx[:, :half], idx[:, half:w]
        a_wins = (a_v > b_v) | ((a_v == b_v) & (a_i < b_i))
        v = jnp.where(a_wins, a_v, b_v)
        idx = jnp.where(a_wins, a_i, b_i)
        col_lt = half // 2 if half > 1 else 1
        j = S // 2
        while j >= 1:
            v, idx = _cswap(v, idx, j, 0, col_lt)
            j //= 2
        w = half

    thr = thr_ref[0]
    mask = v >= thr
    ovals_ref[...] = jnp.where(mask, v, 0.0)
    oidx_ref[...] = idx
    omask_ref[...] = mask.astype(jnp.int32)


def _gather_paths(paths_flat, flat_idx):
    from jax.experimental.pallas import tpu_sc as plsc

    info = plsc.get_sparse_core_info()
    NC, NS = info.num_cores, info.num_subcores
    NW = NC * NS
    B = flat_idx.shape[0]
    b_per_w = B // NW
    mesh = plsc.VectorSubcoreMesh(core_axis_name="c", subcore_axis_name="s")

    @functools.partial(
        pl.kernel, mesh=mesh,
        out_type=jax.ShapeDtypeStruct((B,), jnp.int32),
        scratch_types=[
            pltpu.VMEM((b_per_w,), jnp.int32),
            pltpu.VMEM((b_per_w,), jnp.int32),
            pltpu.SemaphoreType.DMA,
        ],
    )
    def k(table_hbm, idx_hbm, out_hbm, idx_v, rows_v, sem):
        wid = lax.axis_index("s") * NC + lax.axis_index("c")
        base = wid * b_per_w
        pltpu.sync_copy(idx_hbm.at[pl.ds(base, b_per_w)], idx_v)
        pltpu.async_copy(table_hbm.at[idx_v], rows_v, sem).wait()
        pltpu.sync_copy(rows_v, out_hbm.at[pl.ds(base, b_per_w)])

    return k(paths_flat, flat_idx)


def kernel(importances, threshold, paths):
    n = importances.shape[0]
    imp_p = jnp.concatenate(
        [importances, jnp.full((N_PAD - n,), -jnp.inf, jnp.float32)])
    imp_p = imp_p.reshape(S, C)

    svals, sidx = pl.pallas_call(
        _colsort_kernel,
        grid=(NSTEPS,),
        in_specs=[pl.BlockSpec((S, CB), lambda s: (0, s))],
        out_specs=[pl.BlockSpec((S, CB), lambda s: (0, s)),
                   pl.BlockSpec((S, CB), lambda s: (0, s))],
        out_shape=[jax.ShapeDtypeStruct((S, C), jnp.float32),
                   jax.ShapeDtypeStruct((S, C), jnp.int32)],
    )(imp_p)

    thr = jnp.reshape(threshold.astype(jnp.float32), (1,))
    out_vals2, top_idx2, mask2 = pl.pallas_call(
        _fold_kernel,
        in_specs=[pl.BlockSpec(memory_space=pltpu.SMEM),
                  pl.BlockSpec((S, C), lambda: (0, 0)),
                  pl.BlockSpec((S, C), lambda: (0, 0))],
        out_specs=[pl.BlockSpec((K, 1), lambda: (0, 0)),
                   pl.BlockSpec((K, 1), lambda: (0, 0)),
                   pl.BlockSpec((K, 1), lambda: (0, 0))],
        out_shape=[jax.ShapeDtypeStruct((K, 1), jnp.float32),
                   jax.ShapeDtypeStruct((K, 1), jnp.int32),
                   jax.ShapeDtypeStruct((K, 1), jnp.int32)],
    )(thr, svals, sidx)

    out_vals = out_vals2.reshape(K)
    top_idx = top_idx2.reshape(K)
    mask = mask2.reshape(K).astype(jnp.bool_)

    flat_idx = top_idx[:, None] * 3 + jnp.arange(3, dtype=jnp.int32)[None, :]
    flat_idx = jnp.where(mask[:, None], flat_idx, 0).reshape(-1)

    gathered = _gather_paths(paths.reshape(-1), flat_idx)
    sel_paths = jnp.where(mask[:, None], gathered.reshape(K, 3), 0)
    return out_vals, sel_paths, mask
